# baseline (speedup 1.0000x reference)
import functools

import jax
import jax.numpy as jnp
from jax import lax
from jax.experimental import pallas as pl
from jax.experimental.pallas import tpu as pltpu

_B = 128
_SPAD = 256
_TOKENS = _B * (_B - 1) // 2
_D = 1024
_NSTRIPE = 8
_ROWS_PER_STRIPE = _TOKENS // _NSTRIPE


def _fused_body(values_ref, rl_ref, vout_ref, splits_ref, sem):
    for k in range(_NSTRIPE):
        pltpu.make_async_copy(
            values_ref.at[pl.ds(k * _ROWS_PER_STRIPE, _ROWS_PER_STRIPE)],
            vout_ref.at[pl.ds(k * _ROWS_PER_STRIPE, _ROWS_PER_STRIPE)],
            sem,
        ).start()

    rl_col = rl_ref[...]
    j = lax.broadcasted_iota(jnp.int32, (_B, _SPAD), 0)
    i = lax.broadcasted_iota(jnp.int32, (_B, _SPAD), 1)
    contrib = jnp.where(j < i, rl_col, 0)
    splits_ref[...] = jnp.sum(contrib, axis=0, keepdims=True)

    for k in range(_NSTRIPE):
        pltpu.make_async_copy(
            values_ref.at[pl.ds(k * _ROWS_PER_STRIPE, _ROWS_PER_STRIPE)],
            vout_ref.at[pl.ds(k * _ROWS_PER_STRIPE, _ROWS_PER_STRIPE)],
            sem,
        ).wait()


_fused_tc = pl.pallas_call(
    _fused_body,
    in_specs=[
        pl.BlockSpec(memory_space=pl.ANY),
        pl.BlockSpec((_B, 1), lambda: (0, 0)),
    ],
    out_specs=[
        pl.BlockSpec(memory_space=pl.ANY),
        pl.BlockSpec((1, _SPAD), lambda: (0, 0)),
    ],
    out_shape=[
        jax.ShapeDtypeStruct((_TOKENS, _D), jnp.float32),
        jax.ShapeDtypeStruct((1, _SPAD), jnp.int32),
    ],
    scratch_shapes=[pltpu.SemaphoreType.DMA],
)


def kernel(values, row_lengths):
    values_out, splits_pad = _fused_tc(values, row_lengths.reshape(_B, 1))
    row_splits = splits_pad.reshape(_SPAD)[: _B + 1]
    return values_out, row_splits


# speedup vs baseline: 37.8957x; 37.8957x over previous
import jax
import jax.numpy as jnp
from jax import lax
from jax.experimental import pallas as pl

_B = 128
_SPAD = 256
_TOKENS = _B * (_B - 1) // 2
_D = 1024
_BLK = 1016
_CBLK = 512


def _fused_body(values_ref, rl_ref, vout_ref, splits_ref):
    vout_ref[...] = values_ref[...]

    @pl.when((pl.program_id(0) == 0) & (pl.program_id(1) == 0))
    def _():
        rl_col = rl_ref[...]
        j = lax.broadcasted_iota(jnp.int32, (_B, _SPAD), 0)
        i = lax.broadcasted_iota(jnp.int32, (_B, _SPAD), 1)
        contrib = jnp.where(j < i, rl_col, 0)
        splits_ref[...] = jnp.sum(contrib, axis=0, keepdims=True)


_fused_tc = pl.pallas_call(
    _fused_body,
    grid=(_TOKENS // _BLK, _D // _CBLK),
    in_specs=[
        pl.BlockSpec((_BLK, _CBLK), lambda i, j: (i, j)),
        pl.BlockSpec((_B, 1), lambda i, j: (0, 0)),
    ],
    out_specs=[
        pl.BlockSpec((_BLK, _CBLK), lambda i, j: (i, j)),
        pl.BlockSpec((1, _SPAD), lambda i, j: (0, 0)),
    ],
    out_shape=[
        jax.ShapeDtypeStruct((_TOKENS, _D), jnp.float32),
        jax.ShapeDtypeStruct((1, _SPAD), jnp.int32),
    ],
)


def kernel(values, row_lengths):
    values_out, splits_pad = _fused_tc(values, row_lengths.reshape(_B, 1))
    row_splits = splits_pad.reshape(_SPAD)[: _B + 1]
    return values_out, row_splits


# speedup vs baseline: 38.4161x; 1.0137x over previous
import jax
import jax.numpy as jnp
from jax import lax
from jax.experimental import pallas as pl

_B = 128
_SPAD = 256
_TOKENS = _B * (_B - 1) // 2
_D = 1024
_BLK = 512
_CBLK = _D


def _fused_body(values_ref, rl_ref, vout_ref, splits_ref):
    vout_ref[...] = values_ref[...]

    @pl.when((pl.program_id(0) == 0) & (pl.program_id(1) == 0))
    def _():
        rl_col = rl_ref[...]
        j = lax.broadcasted_iota(jnp.int32, (_B, _SPAD), 0)
        i = lax.broadcasted_iota(jnp.int32, (_B, _SPAD), 1)
        contrib = jnp.where(j < i, rl_col, 0)
        splits_ref[...] = jnp.sum(contrib, axis=0, keepdims=True)


_fused_tc = pl.pallas_call(
    _fused_body,
    grid=((_TOKENS + _BLK - 1) // _BLK, _D // _CBLK),
    in_specs=[
        pl.BlockSpec((_BLK, _CBLK), lambda i, j: (i, j)),
        pl.BlockSpec((_B, 1), lambda i, j: (0, 0)),
    ],
    out_specs=[
        pl.BlockSpec((_BLK, _CBLK), lambda i, j: (i, j)),
        pl.BlockSpec((1, _SPAD), lambda i, j: (0, 0)),
    ],
    out_shape=[
        jax.ShapeDtypeStruct((_TOKENS, _D), jnp.float32),
        jax.ShapeDtypeStruct((1, _SPAD), jnp.int32),
    ],
)


def kernel(values, row_lengths):
    values_out, splits_pad = _fused_tc(values, row_lengths.reshape(_B, 1))
    row_splits = splits_pad.reshape(_SPAD)[: _B + 1]
    return values_out, row_splits


# speedup vs baseline: 44.2459x; 1.1518x over previous
import jax
import jax.numpy as jnp
from jax import lax
from jax.experimental import pallas as pl

_B = 128
_SPAD = 256
_TOKENS = _B * (_B - 1) // 2
_D = 1024
_BLK = 1016


def _fused_body(values_ref, rl_ref, vout_ref, splits_ref):
    vout_ref[...] = values_ref[...]

    @pl.when(pl.program_id(0) == 0)
    def _():
        rl_row = rl_ref[...].astype(jnp.float32)
        j = lax.broadcasted_iota(jnp.int32, (_B, _SPAD), 0)
        i = lax.broadcasted_iota(jnp.int32, (_B, _SPAD), 1)
        tri = jnp.where(j < i, 1.0, 0.0)
        splits = jnp.dot(rl_row, tri, preferred_element_type=jnp.float32)
        splits_ref[...] = splits.astype(jnp.int32)


_fused_tc = pl.pallas_call(
    _fused_body,
    grid=(_TOKENS // _BLK,),
    in_specs=[
        pl.BlockSpec((_BLK, _D), lambda i: (i, 0)),
        pl.BlockSpec((1, _B), lambda i: (0, 0)),
    ],
    out_specs=[
        pl.BlockSpec((_BLK, _D), lambda i: (i, 0)),
        pl.BlockSpec((1, _SPAD), lambda i: (0, 0)),
    ],
    out_shape=[
        jax.ShapeDtypeStruct((_TOKENS, _D), jnp.float32),
        jax.ShapeDtypeStruct((1, _SPAD), jnp.int32),
    ],
)


def kernel(values, row_lengths):
    values_out, splits_pad = _fused_tc(values, row_lengths.reshape(1, _B))
    row_splits = splits_pad.reshape(_SPAD)[: _B + 1]
    return values_out, row_splits


# speedup vs baseline: 47.2565x; 1.0680x over previous
import jax
import jax.numpy as jnp
from jax import lax
from jax.experimental import pallas as pl

_B = 128
_SPAD = 256
_TOKENS = _B * (_B - 1) // 2
_D = 1024
_BLK = 2032


def _fused_body(values_ref, rl_ref, vout_ref, splits_ref):
    vout_ref[...] = values_ref[...]

    @pl.when(pl.program_id(0) == 0)
    def _():
        rl_row = rl_ref[...].astype(jnp.float32)
        j = lax.broadcasted_iota(jnp.int32, (_B, _SPAD), 0)
        i = lax.broadcasted_iota(jnp.int32, (_B, _SPAD), 1)
        tri = jnp.where(j < i, 1.0, 0.0)
        splits = jnp.dot(rl_row, tri, preferred_element_type=jnp.float32)
        splits_ref[...] = splits.astype(jnp.int32)


_fused_tc = pl.pallas_call(
    _fused_body,
    grid=(_TOKENS // _BLK,),
    in_specs=[
        pl.BlockSpec((_BLK, _D), lambda i: (i, 0)),
        pl.BlockSpec((1, _B), lambda i: (0, 0)),
    ],
    out_specs=[
        pl.BlockSpec((_BLK, _D), lambda i: (i, 0)),
        pl.BlockSpec((1, _SPAD), lambda i: (0, 0)),
    ],
    out_shape=[
        jax.ShapeDtypeStruct((_TOKENS, _D), jnp.float32),
        jax.ShapeDtypeStruct((1, _SPAD), jnp.int32),
    ],
)


def kernel(values, row_lengths):
    values_out, splits_pad = _fused_tc(values, row_lengths.reshape(1, _B))
    row_splits = splits_pad.reshape(_SPAD)[: _B + 1]
    return values_out, row_splits


# speedup vs baseline: 48.7133x; 1.0308x over previous
import jax
import jax.numpy as jnp
from jax import lax
from jax.experimental import pallas as pl
from jax.experimental.pallas import tpu as pltpu

_B = 128
_SPAD = 256
_TOKENS = _B * (_B - 1) // 2
_D = 1024
_BLK = 4064


def _fused_body(values_ref, rl_ref, vout_ref, splits_ref):
    vout_ref[...] = values_ref[...]

    @pl.when(pl.program_id(0) == 0)
    def _():
        rl_row = rl_ref[...].astype(jnp.float32)
        j = lax.broadcasted_iota(jnp.int32, (_B, _SPAD), 0)
        i = lax.broadcasted_iota(jnp.int32, (_B, _SPAD), 1)
        tri = jnp.where(j < i, 1.0, 0.0)
        splits = jnp.dot(rl_row, tri, preferred_element_type=jnp.float32)
        splits_ref[...] = splits.astype(jnp.int32)


_fused_tc = pl.pallas_call(
    _fused_body,
    grid=(_TOKENS // _BLK,),
    in_specs=[
        pl.BlockSpec((_BLK, _D), lambda i: (i, 0)),
        pl.BlockSpec((1, _B), lambda i: (0, 0)),
    ],
    out_specs=[
        pl.BlockSpec((_BLK, _D), lambda i: (i, 0)),
        pl.BlockSpec((1, _SPAD), lambda i: (0, 0)),
    ],
    out_shape=[
        jax.ShapeDtypeStruct((_TOKENS, _D), jnp.float32),
        jax.ShapeDtypeStruct((1, _SPAD), jnp.int32),
    ],
    compiler_params=pltpu.CompilerParams(vmem_limit_bytes=100 * 1024 * 1024),
)


def kernel(values, row_lengths):
    values_out, splits_pad = _fused_tc(values, row_lengths.reshape(1, _B))
    row_splits = splits_pad.reshape(_SPAD)[: _B + 1]
    return values_out, row_splits
